# baseline (speedup 1.0000x reference)
import functools

import jax
import jax.numpy as jnp
from jax import lax
from jax.experimental import pallas as pl
from jax.experimental.pallas import tpu as pltpu
from jax.experimental.pallas import tpu_sc as plsc

_NC = 2
_NS = 16
_NW = _NC * _NS
_L = 16
_NBUF = 3


def _transe_sc(h_head, h_tail, rels, rel_emb, S):
    B, F = h_head.shape
    bpw = S // _NW
    C = max(c for c in range(16, min(bpw, 128) + 1, 16) if bpw % c == 0)
    nchunks = bpw // C
    nbuf = min(_NBUF, nchunks)
    W = C + 1
    mesh = plsc.VectorSubcoreMesh(core_axis_name="c", subcore_axis_name="s")

    N = rel_emb.shape[0]
    scratch = dict(
        idx_v=pltpu.VMEM((bpw,), jnp.int32),
        out_v=pltpu.VMEM((bpw,), jnp.float32),
        tr_v=pltpu.VMEM((_L * W,), jnp.float32),
        tbl=pltpu.VMEM_SHARED((N, F), jnp.float32),
        isem=pltpu.SemaphoreType.DMA,
        hsem=pltpu.SemaphoreType.DMA((nbuf,)),
        tsem=pltpu.SemaphoreType.DMA((nbuf,)),
        rsem=pltpu.SemaphoreType.DMA((nbuf,)),
    )
    for i in range(nbuf):
        scratch[f"hp{i}"] = pltpu.VMEM((C, F), jnp.float32)
        scratch[f"t{i}"] = pltpu.VMEM((C, F), jnp.float32)

    @functools.partial(
        pl.kernel,
        out_type=jax.ShapeDtypeStruct((B,), jnp.float32),
        mesh=mesh,
        scratch_types=scratch,
        compiler_params=pltpu.CompilerParams(needs_layout_passes=False),
    )
    def k(head_hbm, tail_hbm, rels_hbm, emb_hbm, out_hbm,
          idx_v, out_v, tr_v, tbl, isem, hsem, tsem, rsem, **bufs):
        hp = [bufs[f"hp{i}"] for i in range(nbuf)]
        tb = [bufs[f"t{i}"] for i in range(nbuf)]
        wid = lax.axis_index("s") * _NC + lax.axis_index("c")
        base = wid * bpw
        lanes = lax.iota(jnp.int32, _L)
        lanes_w = lanes * W

        def issue_ht(g):
            s = g % nbuf
            ch = pltpu.async_copy(head_hbm.at[pl.ds(base + g * C, C)],
                                  hp[s], hsem.at[s])
            ct = pltpu.async_copy(tail_hbm.at[pl.ds(base + g * C, C)],
                                  tb[s], tsem.at[s])
            return ch, ct

        def issue_r(g):
            s = g % nbuf
            return pltpu.async_copy(tbl.at[idx_v.at[pl.ds(g * C, C)]],
                                    hp[s], rsem.at[s], add=True)

        cph, cpt, cpr = {}, {}, {}
        cph[0], cpt[0] = issue_ht(0)
        cpi = pltpu.async_copy(rels_hbm.at[pl.ds(base, bpw)], idx_v, isem)

        @pl.when(lax.axis_index("s") == 0)
        def _():
            pltpu.sync_copy(emb_hbm, tbl)

        plsc.subcore_barrier()
        cpi.wait()
        cph[0].wait()
        cpr[0] = issue_r(0)
        if nchunks > 1:
            cph[1], cpt[1] = issue_ht(1)

        for g in range(nchunks):
            if g + 1 < nchunks:
                cph[g + 1].wait()
                cpr[g + 1] = issue_r(g + 1)
            if g + 2 < nchunks:
                cph[g + 2], cpt[g + 2] = issue_ht(g + 2)
            cpr[g].wait()
            cpt[g].wait()
            s = g % nbuf
            hp_g, t_g = hp[s], tb[s]

            @plsc.parallel_loop(0, C, unroll=4)
            def _(r, hp_g=hp_g, t_g=t_g):
                d = [jnp.abs(hp_g[r, pl.ds(v * _L, _L)] -
                             t_g[r, pl.ds(v * _L, _L)])
                     for v in range(F // _L)]
                while len(d) > 1:
                    d = [a + b for a, b in zip(d[::2], d[1::2])]
                plsc.store_scatter(tr_v, [lanes_w + r], d[0])

            def group_body(g2, carry, g=g):
                col0 = g2 * _L
                out_acc = tr_v[pl.ds(col0, _L)]
                for j in range(1, _L):
                    out_acc = out_acc + tr_v[pl.ds(j * W + col0, _L)]
                out_v[pl.ds(g * C + g2 * _L, _L)] = -out_acc
                return carry

            lax.fori_loop(0, C // _L, group_body, 0)
        pltpu.sync_copy(out_v, out_hbm.at[pl.ds(base, bpw)])

    return k(h_head, h_tail, rels, rel_emb)


_TC_BLK = 1024


def _transe_tc(h_head, h_tail, rels, rel_emb, S):
    B, F = h_head.shape
    NT = rel_emb.shape[0]
    grid = (B - S) // _TC_BLK
    off = S // _TC_BLK
    rels3 = rels.reshape(B // _TC_BLK, 1, _TC_BLK)

    def body(h_ref, t_ref, r_ref, e_ref, o_ref):
        r_blk = r_ref[0, 0, :]
        col = jax.lax.broadcasted_iota(jnp.int32, (_TC_BLK, NT), 1)
        onehot = (r_blk[:, None] == col).astype(jnp.bfloat16)
        rel = jnp.dot(onehot, e_ref[...].astype(jnp.bfloat16),
                      preferred_element_type=jnp.float32)
        score = jnp.sum(jnp.abs(h_ref[...] + rel - t_ref[...]), axis=1)
        o_ref[0, 0, :] = -score

    out = pl.pallas_call(
        body,
        grid=(grid,),
        in_specs=[
            pl.BlockSpec((_TC_BLK, F), lambda i: (i + off, 0)),
            pl.BlockSpec((_TC_BLK, F), lambda i: (i + off, 0)),
            pl.BlockSpec((1, 1, _TC_BLK), lambda i: (i + off, 0, 0)),
            pl.BlockSpec((NT, F), lambda i: (0, 0)),
        ],
        out_specs=pl.BlockSpec((1, 1, _TC_BLK), lambda i: (i, 0, 0)),
        out_shape=jax.ShapeDtypeStruct((grid, 1, _TC_BLK), jnp.float32),
    )(h_head, h_tail, rels3, rel_emb)
    return out.reshape(B - S)


_SC_ROWS = 11264


def kernel(h_head, h_tail, rels, rel_emb):
    rels = rels.astype(jnp.int32)
    out_sc = _transe_sc(h_head, h_tail, rels, rel_emb, _SC_ROWS)
    out_tc = _transe_tc(h_head, h_tail, rels, rel_emb, _SC_ROWS)
    return jax.lax.dynamic_update_slice(out_sc, out_tc, (_SC_ROWS,))


# speedup vs baseline: 1.1746x; 1.1746x over previous
import functools

import jax
import jax.numpy as jnp
from jax import lax
from jax.experimental import pallas as pl
from jax.experimental.pallas import tpu as pltpu
from jax.experimental.pallas import tpu_sc as plsc

_NC = 2
_NS = 16
_NW = _NC * _NS
_L = 16
_NBUF = 3


def _transe_sc(h_head, h_tail, rels, rel_emb, S):
    B, F = h_head.shape
    bpw = S // _NW
    C = max(c for c in range(16, min(bpw, 160) + 1, 16) if bpw % c == 0)
    nchunks = bpw // C
    nbuf = min(_NBUF, nchunks)
    W = C + 1
    mesh = plsc.VectorSubcoreMesh(core_axis_name="c", subcore_axis_name="s")

    N = rel_emb.shape[0]
    scratch = dict(
        idx_v=pltpu.VMEM((bpw,), jnp.int32),
        out_v=pltpu.VMEM((bpw,), jnp.float32),
        tr_v=pltpu.VMEM((_L * W,), jnp.float32),
        tbl=pltpu.VMEM_SHARED((N, F), jnp.float32),
        isem=pltpu.SemaphoreType.DMA,
        hsem=pltpu.SemaphoreType.DMA((nbuf,)),
        tsem=pltpu.SemaphoreType.DMA((nbuf,)),
        rsem=pltpu.SemaphoreType.DMA((nbuf,)),
    )
    for i in range(nbuf):
        scratch[f"hp{i}"] = pltpu.VMEM((C, F), jnp.float32)
        scratch[f"t{i}"] = pltpu.VMEM((C, F), jnp.float32)

    @functools.partial(
        pl.kernel,
        out_type=jax.ShapeDtypeStruct((B,), jnp.float32),
        mesh=mesh,
        scratch_types=scratch,
        compiler_params=pltpu.CompilerParams(needs_layout_passes=False),
    )
    def k(head_hbm, tail_hbm, rels_hbm, emb_hbm, out_hbm,
          idx_v, out_v, tr_v, tbl, isem, hsem, tsem, rsem, **bufs):
        hp = [bufs[f"hp{i}"] for i in range(nbuf)]
        tb = [bufs[f"t{i}"] for i in range(nbuf)]
        wid = lax.axis_index("s") * _NC + lax.axis_index("c")
        base = wid * bpw
        lanes = lax.iota(jnp.int32, _L)
        lanes_w = lanes * W

        def issue_ht(g):
            s = g % nbuf
            ch = pltpu.async_copy(head_hbm.at[pl.ds(base + g * C, C)],
                                  hp[s], hsem.at[s])
            ct = pltpu.async_copy(tail_hbm.at[pl.ds(base + g * C, C)],
                                  tb[s], tsem.at[s])
            return ch, ct

        def issue_r(g):
            s = g % nbuf
            return pltpu.async_copy(tbl.at[idx_v.at[pl.ds(g * C, C)]],
                                    hp[s], rsem.at[s], add=True)

        cph, cpt, cpr = {}, {}, {}
        cph[0], cpt[0] = issue_ht(0)
        cpi = pltpu.async_copy(rels_hbm.at[pl.ds(base, bpw)], idx_v, isem)

        @pl.when(lax.axis_index("s") == 0)
        def _():
            pltpu.sync_copy(emb_hbm, tbl)

        plsc.subcore_barrier()
        cpi.wait()
        cph[0].wait()
        cpr[0] = issue_r(0)
        if nchunks > 1:
            cph[1], cpt[1] = issue_ht(1)

        for g in range(nchunks):
            if g + 1 < nchunks:
                cph[g + 1].wait()
                cpr[g + 1] = issue_r(g + 1)
            if g + 2 < nchunks:
                cph[g + 2], cpt[g + 2] = issue_ht(g + 2)
            cpr[g].wait()
            cpt[g].wait()
            s = g % nbuf
            hp_g, t_g = hp[s], tb[s]

            @plsc.parallel_loop(0, C, unroll=4)
            def _(r, hp_g=hp_g, t_g=t_g):
                d = [jnp.abs(hp_g[r, pl.ds(v * _L, _L)] -
                             t_g[r, pl.ds(v * _L, _L)])
                     for v in range(F // _L)]
                while len(d) > 1:
                    d = [a + b for a, b in zip(d[::2], d[1::2])]
                plsc.store_scatter(tr_v, [lanes_w + r], d[0])

            def group_body(g2, carry, g=g):
                col0 = g2 * _L
                out_acc = tr_v[pl.ds(col0, _L)]
                for j in range(1, _L):
                    out_acc = out_acc + tr_v[pl.ds(j * W + col0, _L)]
                out_v[pl.ds(g * C + g2 * _L, _L)] = -out_acc
                return carry

            lax.fori_loop(0, C // _L, group_body, 0)
        pltpu.sync_copy(out_v, out_hbm.at[pl.ds(base, bpw)])

    return k(h_head, h_tail, rels, rel_emb)


_TC_BLK = 1024


def _transe_tc(h_head, h_tail, rels, rel_emb, S):
    B, F = h_head.shape
    NT = rel_emb.shape[0]
    grid = (B - S) // _TC_BLK
    off = S // _TC_BLK
    rels3 = rels.reshape(B // _TC_BLK, 1, _TC_BLK)

    def body(h_ref, t_ref, r_ref, e_ref, o_ref):
        r_blk = r_ref[0, 0, :]
        col = jax.lax.broadcasted_iota(jnp.int32, (_TC_BLK, NT), 1)
        onehot = (r_blk[:, None] == col).astype(jnp.bfloat16)
        rel = jnp.dot(onehot, e_ref[...].astype(jnp.bfloat16),
                      preferred_element_type=jnp.float32)
        score = jnp.sum(jnp.abs(h_ref[...] + rel - t_ref[...]), axis=1)
        o_ref[0, 0, :] = -score

    out = pl.pallas_call(
        body,
        grid=(grid,),
        in_specs=[
            pl.BlockSpec((_TC_BLK, F), lambda i: (i + off, 0)),
            pl.BlockSpec((_TC_BLK, F), lambda i: (i + off, 0)),
            pl.BlockSpec((1, 1, _TC_BLK), lambda i: (i + off, 0, 0)),
            pl.BlockSpec((NT, F), lambda i: (0, 0)),
        ],
        out_specs=pl.BlockSpec((1, 1, _TC_BLK), lambda i: (i, 0, 0)),
        out_shape=jax.ShapeDtypeStruct((grid, 1, _TC_BLK), jnp.float32),
    )(h_head, h_tail, rels3, rel_emb)
    return out.reshape(B - S)


_SC_ROWS = 10240


def kernel(h_head, h_tail, rels, rel_emb):
    rels = rels.astype(jnp.int32)
    out_sc = _transe_sc(h_head, h_tail, rels, rel_emb, _SC_ROWS)
    out_tc = _transe_tc(h_head, h_tail, rels, rel_emb, _SC_ROWS)
    return jax.lax.dynamic_update_slice(out_sc, out_tc, (_SC_ROWS,))
